# baseline (speedup 1.0000x reference)
import functools

import jax
import jax.numpy as jnp
import numpy as np
from jax import lax
from jax.experimental import pallas as pl
from jax.experimental.pallas import tpu as pltpu
from jax.experimental.pallas import tpu_sc as plsc

N = 50000
E = 800000
F = 64
HF = 32
G = 20
BN = 128
FFN = 128
KOUT = 160000

NSC = 2
NT = 16
CH = 128
NCH = 56
BLKE = NCH * CH

EPT = 50176
CPT = EPT // CH
NBLK = CPT // NCH
EP = NT * EPT
SR = 8
SLAB = SR * CH
NSL = NCH // SR

NP1 = 50176
ZPT = NP1 // NT
ZR = 392

_mesh = plsc.VectorSubcoreMesh(core_axis_name="c", subcore_axis_name="s")
_sc_params = pltpu.CompilerParams(
    needs_layout_passes=False, use_tc_tiling_on_sc=False
)


def _elu(v):
    return jnp.where(v > 0, v, jnp.exp(v) - 1.0)


@functools.partial(
    pl.kernel,
    out_type=jax.ShapeDtypeStruct((NSC * NP1,), jnp.float32),
    mesh=_mesh,
    scratch_types=[
        pltpu.VMEM((NCH, CH), jnp.int32),
        pltpu.VMEM((BLKE,), jnp.float32),
        pltpu.VMEM((ZPT,), jnp.float32),
        pltpu.VMEM_SHARED((NP1,), jnp.float32),
    ],
)
def _deg_sc(col2_hbm, w_hbm, out_hbm, col_v, w_v, z_v, acc_sh):
    c = lax.axis_index("c")
    s = lax.axis_index("s")
    z16 = jnp.zeros((16,), jnp.float32)

    def zfill(i, _):
        z_v[pl.ds(i * 16, 16)] = z16
        return 0

    lax.fori_loop(0, ZPT // 16, zfill, 0)
    pltpu.sync_copy(z_v, acc_sh.at[pl.ds(s * ZPT, ZPT)])
    plsc.subcore_barrier()

    def blk(b, _):
        crb = s * CPT + b * NCH
        pltpu.sync_copy(col2_hbm.at[pl.ds(crb, NCH)], col_v)
        pltpu.sync_copy(w_hbm.at[pl.ds(crb * CH, BLKE)], w_v)

        def ch(j, _):
            pltpu.sync_copy(
                w_v.at[pl.ds(j * CH, CH)], acc_sh.at[col_v.at[j]], add=True
            )
            return 0

        lax.fori_loop(0, NCH, ch, 0)
        return 0

    lax.fori_loop(0, NBLK, blk, 0)
    plsc.subcore_barrier()
    pltpu.sync_copy(acc_sh.at[pl.ds(s * ZPT, ZPT)], z_v)
    pltpu.sync_copy(z_v, out_hbm.at[pl.ds(c * NP1 + s * ZPT, ZPT)])


QF = 16

_GDN = lax.GatherDimensionNumbers(
    offset_dims=(), collapsed_slice_dims=(0,), start_index_map=(0,)
)


@functools.partial(
    pl.kernel,
    out_type=jax.ShapeDtypeStruct((4, NP1, QF), jnp.float32),
    mesh=_mesh,
    scratch_types=[
        pltpu.VMEM((BLKE,), jnp.int32),
        pltpu.VMEM((BLKE,), jnp.int32),
        pltpu.VMEM((BLKE,), jnp.float32),
        pltpu.VMEM((SLAB, QF), jnp.float32),
        pltpu.VMEM((SLAB, QF), jnp.float32),
        pltpu.VMEM((ZR, QF), jnp.float32),
        pltpu.VMEM_SHARED((NP1, QF), jnp.float32),
        pltpu.SemaphoreType.DMA,
        pltpu.SemaphoreType.DMA,
        pltpu.SemaphoreType.DMA,
        pltpu.SemaphoreType.DMA,
    ],
    compiler_params=_sc_params,
)
def _conv_sc(xs_hbm, row1_hbm, col1_hbm, w_hbm, out_hbm,
             row_v, col_v, w_v, g_a, g_b, z_v, acc_sh,
             sem_a, sem_b, sem_sa, sem_sb):
    c = lax.axis_index("c")
    s = lax.axis_index("s")
    z16 = jnp.zeros((16,), jnp.float32)

    def zfill(i, _):
        z_v[i, pl.ds(0, 16)] = z16
        return 0

    def sweep(p, _):
        q = c * 2 + p
        lax.fori_loop(0, ZR, zfill, 0)

        def zcopy(k, _):
            pltpu.sync_copy(z_v, acc_sh.at[pl.ds(s * ZPT + k * ZR, ZR)])
            return 0

        lax.fori_loop(0, ZPT // ZR, zcopy, 0)
        plsc.subcore_barrier()

        qNv = jnp.full((16,), q * N, jnp.int32)

        def blk(b, _):
            eb = (s * CPT + b * NCH) * CH
            pltpu.sync_copy(row1_hbm.at[pl.ds(eb, BLKE)], row_v)
            pltpu.sync_copy(col1_hbm.at[pl.ds(eb, BLKE)], col_v)
            pltpu.sync_copy(w_hbm.at[pl.ds(eb, BLKE)], w_v)

            def adj(i, _):
                sl = pl.ds(i * 16, 16)
                row_v[sl] = row_v[sl] + qNv
                return 0

            lax.fori_loop(0, BLKE // 16, adj, 0, unroll=8)

            def scale(j, g):
                def grp(g8, _):
                    w16 = w_v[pl.ds(j * SLAB + g8 * 16, 16)]
                    e0 = g8 * 16
                    for k in range(16):
                        wb = lax.gather(
                            w16, jnp.full((16, 1), k, jnp.int32), _GDN, (1,),
                            mode=lax.GatherScatterMode.PROMISE_IN_BOUNDS,
                        )
                        g[e0 + k, pl.ds(0, 16)] = g[e0 + k, pl.ds(0, 16)] * wb
                    return 0

                lax.fori_loop(0, SLAB // 16, grp, 0)

            def gather(j, g, sem):
                pltpu.async_copy(
                    xs_hbm.at[row_v.at[pl.ds(j * SLAB, SLAB)]], g, sem
                )

            def gwait(j, g, sem):
                pltpu.make_async_copy(
                    xs_hbm.at[row_v.at[pl.ds(j * SLAB, SLAB)]], g, sem
                ).wait()

            def scatter(j, g, sem):
                pltpu.async_copy(
                    g, acc_sh.at[col_v.at[pl.ds(j * SLAB, SLAB)]], sem, add=True
                )

            def swait(j, g, sem):
                pltpu.make_async_copy(
                    g, acc_sh.at[col_v.at[pl.ds(j * SLAB, SLAB)]], sem
                ).wait()

            gather(0, g_a, sem_a)

            def slab_loop(j, _):
                @pl.when(j % 2 == 0)
                def _():
                    gwait(j, g_a, sem_a)

                    @pl.when(j < NSL - 1)
                    def _():
                        @pl.when(j >= 1)
                        def _():
                            swait(j - 1, g_b, sem_sb)

                        gather(j + 1, g_b, sem_b)

                    scale(j, g_a)
                    scatter(j, g_a, sem_sa)

                @pl.when(j % 2 == 1)
                def _():
                    gwait(j, g_b, sem_b)

                    @pl.when(j < NSL - 1)
                    def _():
                        swait(j - 1, g_a, sem_sa)
                        gather(j + 1, g_a, sem_a)

                    scale(j, g_b)
                    scatter(j, g_b, sem_sb)

                return 0

            lax.fori_loop(0, NSL, slab_loop, 0)
            swait(NSL - 2, g_b, sem_sb)
            swait(NSL - 1, g_a, sem_sa)
            return 0

        lax.fori_loop(0, NBLK, blk, 0)
        plsc.subcore_barrier()

        def ocopy(k, _):
            sl = pl.ds(s * ZPT + k * ZR, ZR)
            pltpu.sync_copy(acc_sh.at[sl], z_v)
            pltpu.sync_copy(z_v, out_hbm.at[q, sl])
            return 0

        lax.fori_loop(0, ZPT // ZR, ocopy, 0)
        return 0

    lax.fori_loop(0, 2, sweep, 0)


_BW = 1280
_R = 1024
_NRB = NP1 // _R


def _dense_body(x_ref, W1_ref, b1_ref, W2_ref, b2_ref, o_ref, h0_ref):
    @pl.when(pl.program_id(0) == 0)
    def _():
        h0 = jnp.dot(x_ref[...], W1_ref[...], preferred_element_type=jnp.float32)
        h0_ref[...] = _elu(h0 + b1_ref[...])

    h = jnp.dot(h0_ref[...], W2_ref[...], preferred_element_type=jnp.float32)
    o_ref[...] = _elu(h + b2_ref[...])


_dense_tc = pl.pallas_call(
    _dense_body,
    grid=(KOUT // _BW,),
    in_specs=[
        pl.BlockSpec((G, BN), lambda i: (0, 0)),
        pl.BlockSpec((BN, FFN), lambda i: (0, 0)),
        pl.BlockSpec((FFN,), lambda i: (0,)),
        pl.BlockSpec((FFN, _BW), lambda i: (0, i)),
        pl.BlockSpec((1, _BW), lambda i: (0, i)),
    ],
    out_specs=pl.BlockSpec((G, _BW), lambda i: (0, i)),
    out_shape=jax.ShapeDtypeStruct((G, KOUT), jnp.float32),
    scratch_shapes=[pltpu.VMEM((G, FFN), jnp.float32)],
)


def _dis_block(deg_ref):
    dis = lax.rsqrt(1.0 + deg_ref[...])
    return lax.broadcast_in_dim(dis, (_R, F), (0,))


def _split_q(o_ref, xs):
    for q in range(4):
        o_ref[q, :, :] = xs[:, q * QF:(q + 1) * QF]


def _cat_q(ref):
    return jnp.concatenate([ref[q] for q in range(4)], axis=1)


def _pre_body(deg_ref, h_ref, Wc_ref, o_ref):
    disb = _dis_block(deg_ref)
    xh = jnp.dot(h_ref[...], Wc_ref[...], preferred_element_type=jnp.float32)
    _split_q(o_ref, disb * xh)


_pre_tc = pl.pallas_call(
    _pre_body,
    grid=(_NRB,),
    in_specs=[
        pl.BlockSpec((_R,), lambda i: (i,)),
        pl.BlockSpec((_R, F), lambda i: (i, 0)),
        pl.BlockSpec((F, F), lambda i: (0, 0)),
    ],
    out_specs=pl.BlockSpec((4, _R, QF), lambda i: (0, i, 0)),
    out_shape=jax.ShapeDtypeStruct((4, N, QF), jnp.float32),
)


def _mid_body(p_ref, xs_ref, deg_ref, bc_ref, h_ref, Wc_ref, o_ref):
    disb = _dis_block(deg_ref)
    acc = _cat_q(p_ref)
    xs = _cat_q(xs_ref)
    y = _elu(disb * (acc + xs) + bc_ref[...]) + h_ref[...]
    xh2 = jnp.dot(y, Wc_ref[...], preferred_element_type=jnp.float32)
    _split_q(o_ref, disb * xh2)


_mid_tc = pl.pallas_call(
    _mid_body,
    grid=(_NRB,),
    in_specs=[
        pl.BlockSpec((4, _R, QF), lambda i: (0, i, 0)),
        pl.BlockSpec((4, _R, QF), lambda i: (0, i, 0)),
        pl.BlockSpec((_R,), lambda i: (i,)),
        pl.BlockSpec((F,), lambda i: (0,)),
        pl.BlockSpec((_R, F), lambda i: (i, 0)),
        pl.BlockSpec((F, F), lambda i: (0, 0)),
    ],
    out_specs=pl.BlockSpec((4, _R, QF), lambda i: (0, i, 0)),
    out_shape=jax.ShapeDtypeStruct((4, N, QF), jnp.float32),
)


def _post_body(p_ref, xs_ref, deg_ref, bc_ref, h_ref, o_ref):
    disb = _dis_block(deg_ref)
    acc = _cat_q(p_ref)
    xs = _cat_q(xs_ref)
    o_ref[...] = disb * (acc + xs) + bc_ref[...] + h_ref[...]


_post_tc = pl.pallas_call(
    _post_body,
    grid=(_NRB,),
    in_specs=[
        pl.BlockSpec((4, _R, QF), lambda i: (0, i, 0)),
        pl.BlockSpec((4, _R, QF), lambda i: (0, i, 0)),
        pl.BlockSpec((_R,), lambda i: (i,)),
        pl.BlockSpec((F,), lambda i: (0,)),
        pl.BlockSpec((_R, F), lambda i: (i, 0)),
    ],
    out_specs=pl.BlockSpec((_R, F), lambda i: (i, 0)),
    out_shape=jax.ShapeDtypeStruct((N, F), jnp.float32),
)


def kernel(x, edge_index, edge_weight, W1, b1, W2, b2, Wc1, bc1, Wc2, bc2,
           num_graphs):
    del num_graphs
    ei = edge_index.astype(jnp.int32)
    pad = EP - E
    rowp = jnp.concatenate([ei[0], jnp.zeros((pad,), jnp.int32)])
    colp = jnp.concatenate([ei[1], jnp.zeros((pad,), jnp.int32)])
    wp = jnp.concatenate([edge_weight, jnp.zeros((pad,), jnp.float32)])
    col2 = colp.reshape(EP // CH, CH)

    degp = _deg_sc(col2, wp)
    H = _dense_tc(x, W1, b1, W2, b2.reshape(1, KOUT)).reshape(N, F)
    X1 = _pre_tc(degp, H, Wc1)
    P1 = _conv_sc(X1.reshape(4 * N, QF), rowp, colp, wp)
    X2 = _mid_tc(P1, X1, degp, bc1, H, Wc2)
    P2 = _conv_sc(X2.reshape(4 * N, QF), rowp, colp, wp)
    return _post_tc(P2, X2, degp, bc2, H)


# speedup vs baseline: 1.0304x; 1.0304x over previous
import functools

import jax
import jax.numpy as jnp
import numpy as np
from jax import lax
from jax.experimental import pallas as pl
from jax.experimental.pallas import tpu as pltpu
from jax.experimental.pallas import tpu_sc as plsc

N = 50000
E = 800000
F = 64
HF = 32
G = 20
BN = 128
FFN = 128
KOUT = 160000

NSC = 2
NT = 16
CH = 128
NCH = 56
BLKE = NCH * CH

EPT = 50176
CPT = EPT // CH
NBLK = CPT // NCH
EP = NT * EPT
SR = 8
SLAB = SR * CH
NSL = NCH // SR

NP1 = 50176
ZPT = NP1 // NT
ZR = 392

_mesh = plsc.VectorSubcoreMesh(core_axis_name="c", subcore_axis_name="s")
_sc_params = pltpu.CompilerParams(
    needs_layout_passes=False, use_tc_tiling_on_sc=False
)


def _elu(v):
    return jnp.where(v > 0, v, jnp.exp(v) - 1.0)


@functools.partial(
    pl.kernel,
    out_type=jax.ShapeDtypeStruct((NSC * NP1,), jnp.float32),
    mesh=_mesh,
    scratch_types=[
        pltpu.VMEM((NCH, CH), jnp.int32),
        pltpu.VMEM((BLKE,), jnp.float32),
        pltpu.VMEM((ZPT,), jnp.float32),
        pltpu.VMEM_SHARED((NP1,), jnp.float32),
    ],
)
def _deg_sc(col2_hbm, w_hbm, out_hbm, col_v, w_v, z_v, acc_sh):
    c = lax.axis_index("c")
    s = lax.axis_index("s")
    z16 = jnp.zeros((16,), jnp.float32)

    def zfill(i, _):
        z_v[pl.ds(i * 16, 16)] = z16
        return 0

    lax.fori_loop(0, ZPT // 16, zfill, 0)
    pltpu.sync_copy(z_v, acc_sh.at[pl.ds(s * ZPT, ZPT)])
    plsc.subcore_barrier()

    def blk(b, _):
        crb = s * CPT + b * NCH
        pltpu.sync_copy(col2_hbm.at[pl.ds(crb, NCH)], col_v)
        pltpu.sync_copy(w_hbm.at[pl.ds(crb * CH, BLKE)], w_v)

        def ch(j, _):
            pltpu.sync_copy(
                w_v.at[pl.ds(j * CH, CH)], acc_sh.at[col_v.at[j]], add=True
            )
            return 0

        lax.fori_loop(0, NCH, ch, 0)
        return 0

    lax.fori_loop(0, NBLK, blk, 0)
    plsc.subcore_barrier()
    pltpu.sync_copy(acc_sh.at[pl.ds(s * ZPT, ZPT)], z_v)
    pltpu.sync_copy(z_v, out_hbm.at[pl.ds(c * NP1 + s * ZPT, ZPT)])


QF = 16

_GDN = lax.GatherDimensionNumbers(
    offset_dims=(), collapsed_slice_dims=(0,), start_index_map=(0,)
)


@functools.partial(
    pl.kernel,
    out_type=jax.ShapeDtypeStruct((4, NP1, QF), jnp.float32),
    mesh=_mesh,
    scratch_types=[
        pltpu.VMEM((BLKE,), jnp.int32),
        pltpu.VMEM((BLKE,), jnp.int32),
        pltpu.VMEM((BLKE,), jnp.float32),
        pltpu.VMEM((SLAB, QF), jnp.float32),
        pltpu.VMEM((SLAB, QF), jnp.float32),
        pltpu.VMEM((ZR, QF), jnp.float32),
        pltpu.VMEM_SHARED((NP1, QF), jnp.float32),
        pltpu.SemaphoreType.DMA,
        pltpu.SemaphoreType.DMA,
        pltpu.SemaphoreType.DMA,
        pltpu.SemaphoreType.DMA,
    ],
    compiler_params=_sc_params,
)
def _conv_sc(xs_hbm, row1_hbm, col1_hbm, w_hbm, out_hbm,
             row_v, col_v, w_v, g_a, g_b, z_v, acc_sh,
             sem_a, sem_b, sem_sa, sem_sb):
    c = lax.axis_index("c")
    s = lax.axis_index("s")
    z16 = jnp.zeros((16,), jnp.float32)

    def zfill(i, _):
        z_v[i, pl.ds(0, 16)] = z16
        return 0

    def sweep(p, _):
        q = c * 2 + p
        lax.fori_loop(0, ZR, zfill, 0)

        def zcopy(k, _):
            pltpu.sync_copy(z_v, acc_sh.at[pl.ds(s * ZPT + k * ZR, ZR)])
            return 0

        lax.fori_loop(0, ZPT // ZR, zcopy, 0)
        plsc.subcore_barrier()

        qNv = jnp.full((16,), q * N, jnp.int32)

        def blk(b, _):
            eb = (s * CPT + b * NCH) * CH
            pltpu.sync_copy(row1_hbm.at[pl.ds(eb, BLKE)], row_v)
            pltpu.sync_copy(col1_hbm.at[pl.ds(eb, BLKE)], col_v)
            pltpu.sync_copy(w_hbm.at[pl.ds(eb, BLKE)], w_v)

            def adj(i, _):
                sl = pl.ds(i * 16, 16)
                row_v[sl] = row_v[sl] + qNv
                return 0

            lax.fori_loop(0, BLKE // 16, adj, 0, unroll=8)

            def scale(j, g):
                def grp(g8, _):
                    w16 = w_v[pl.ds(j * SLAB + g8 * 16, 16)]
                    e0 = g8 * 16
                    for k in range(16):
                        wb = lax.gather(
                            w16, jnp.full((16, 1), k, jnp.int32), _GDN, (1,),
                            mode=lax.GatherScatterMode.PROMISE_IN_BOUNDS,
                        )
                        g[e0 + k, pl.ds(0, 16)] = g[e0 + k, pl.ds(0, 16)] * wb
                    return 0

                lax.fori_loop(0, SLAB // 16, grp, 0)

            def gather(j, g, sem):
                pltpu.async_copy(
                    xs_hbm.at[row_v.at[pl.ds(j * SLAB, SLAB)]], g, sem
                )

            def gwait(j, g, sem):
                pltpu.make_async_copy(
                    xs_hbm.at[row_v.at[pl.ds(j * SLAB, SLAB)]], g, sem
                ).wait()

            def scatter(j, g, sem):
                pltpu.async_copy(
                    g, acc_sh.at[col_v.at[pl.ds(j * SLAB, SLAB)]], sem, add=True
                )

            def swait(j, g, sem):
                pltpu.make_async_copy(
                    g, acc_sh.at[col_v.at[pl.ds(j * SLAB, SLAB)]], sem
                ).wait()

            gather(0, g_a, sem_a)

            def slab_loop(j, _):
                @pl.when(j % 2 == 0)
                def _():
                    gwait(j, g_a, sem_a)

                    @pl.when(j < NSL - 1)
                    def _():
                        @pl.when(j >= 1)
                        def _():
                            swait(j - 1, g_b, sem_sb)

                        gather(j + 1, g_b, sem_b)

                    scale(j, g_a)
                    scatter(j, g_a, sem_sa)

                @pl.when(j % 2 == 1)
                def _():
                    gwait(j, g_b, sem_b)

                    @pl.when(j < NSL - 1)
                    def _():
                        swait(j - 1, g_a, sem_sa)
                        gather(j + 1, g_a, sem_a)

                    scale(j, g_b)
                    scatter(j, g_b, sem_sb)

                return 0

            lax.fori_loop(0, NSL, slab_loop, 0)
            swait(NSL - 2, g_b, sem_sb)
            swait(NSL - 1, g_a, sem_sa)
            return 0

        lax.fori_loop(0, NBLK, blk, 0)
        plsc.subcore_barrier()

        def ocopy(k, _):
            sl = pl.ds(s * ZPT + k * ZR, ZR)
            pltpu.sync_copy(acc_sh.at[sl], z_v)
            pltpu.sync_copy(z_v, out_hbm.at[q, sl])
            return 0

        lax.fori_loop(0, ZPT // ZR, ocopy, 0)
        return 0

    lax.fori_loop(0, 2, sweep, 0)


_BW = 1280
_R = 2048
_NRB = (N + _R - 1) // _R


def _dense_body(x_ref, W1_ref, b1_ref, W2_ref, b2_ref, o_ref, h0_ref):
    @pl.when(pl.program_id(0) == 0)
    def _():
        h0 = jnp.dot(x_ref[...], W1_ref[...], preferred_element_type=jnp.float32)
        h0_ref[...] = _elu(h0 + b1_ref[...])

    h = jnp.dot(h0_ref[...], W2_ref[...], preferred_element_type=jnp.float32)
    o_ref[...] = _elu(h + b2_ref[...])


_dense_tc = pl.pallas_call(
    _dense_body,
    grid=(KOUT // _BW,),
    in_specs=[
        pl.BlockSpec((G, BN), lambda i: (0, 0)),
        pl.BlockSpec((BN, FFN), lambda i: (0, 0)),
        pl.BlockSpec((FFN,), lambda i: (0,)),
        pl.BlockSpec((FFN, _BW), lambda i: (0, i)),
        pl.BlockSpec((1, _BW), lambda i: (0, i)),
    ],
    out_specs=pl.BlockSpec((G, _BW), lambda i: (0, i)),
    out_shape=jax.ShapeDtypeStruct((G, KOUT), jnp.float32),
    scratch_shapes=[pltpu.VMEM((G, FFN), jnp.float32)],
)


def _dis_block(deg_ref):
    dis = lax.rsqrt(1.0 + deg_ref[...])
    return lax.broadcast_in_dim(dis, (_R, F), (0,))


def _split_q(o_ref, xs):
    for q in range(4):
        o_ref[q, :, :] = xs[:, q * QF:(q + 1) * QF]


def _cat_q(ref):
    return jnp.concatenate([ref[q] for q in range(4)], axis=1)


def _pre_body(deg_ref, h_ref, Wc_ref, o_ref):
    disb = _dis_block(deg_ref)
    xh = jnp.dot(h_ref[...], Wc_ref[...], preferred_element_type=jnp.float32)
    _split_q(o_ref, disb * xh)


_pre_tc = pl.pallas_call(
    _pre_body,
    grid=(_NRB,),
    in_specs=[
        pl.BlockSpec((_R,), lambda i: (i,)),
        pl.BlockSpec((_R, F), lambda i: (i, 0)),
        pl.BlockSpec((F, F), lambda i: (0, 0)),
    ],
    out_specs=pl.BlockSpec((4, _R, QF), lambda i: (0, i, 0)),
    out_shape=jax.ShapeDtypeStruct((4, N, QF), jnp.float32),
)


def _mid_body(p_ref, xs_ref, deg_ref, bc_ref, h_ref, Wc_ref, o_ref):
    disb = _dis_block(deg_ref)
    acc = _cat_q(p_ref)
    xs = _cat_q(xs_ref)
    y = _elu(disb * (acc + xs) + bc_ref[...]) + h_ref[...]
    xh2 = jnp.dot(y, Wc_ref[...], preferred_element_type=jnp.float32)
    _split_q(o_ref, disb * xh2)


_mid_tc = pl.pallas_call(
    _mid_body,
    grid=(_NRB,),
    in_specs=[
        pl.BlockSpec((4, _R, QF), lambda i: (0, i, 0)),
        pl.BlockSpec((4, _R, QF), lambda i: (0, i, 0)),
        pl.BlockSpec((_R,), lambda i: (i,)),
        pl.BlockSpec((F,), lambda i: (0,)),
        pl.BlockSpec((_R, F), lambda i: (i, 0)),
        pl.BlockSpec((F, F), lambda i: (0, 0)),
    ],
    out_specs=pl.BlockSpec((4, _R, QF), lambda i: (0, i, 0)),
    out_shape=jax.ShapeDtypeStruct((4, N, QF), jnp.float32),
)


def _post_body(p_ref, xs_ref, deg_ref, bc_ref, h_ref, o_ref):
    disb = _dis_block(deg_ref)
    acc = _cat_q(p_ref)
    xs = _cat_q(xs_ref)
    o_ref[...] = disb * (acc + xs) + bc_ref[...] + h_ref[...]


_post_tc = pl.pallas_call(
    _post_body,
    grid=(_NRB,),
    in_specs=[
        pl.BlockSpec((4, _R, QF), lambda i: (0, i, 0)),
        pl.BlockSpec((4, _R, QF), lambda i: (0, i, 0)),
        pl.BlockSpec((_R,), lambda i: (i,)),
        pl.BlockSpec((F,), lambda i: (0,)),
        pl.BlockSpec((_R, F), lambda i: (i, 0)),
    ],
    out_specs=pl.BlockSpec((_R, F), lambda i: (i, 0)),
    out_shape=jax.ShapeDtypeStruct((N, F), jnp.float32),
)


def kernel(x, edge_index, edge_weight, W1, b1, W2, b2, Wc1, bc1, Wc2, bc2,
           num_graphs):
    del num_graphs
    ei = edge_index.astype(jnp.int32)
    pad = EP - E
    rowp = jnp.concatenate([ei[0], jnp.zeros((pad,), jnp.int32)])
    colp = jnp.concatenate([ei[1], jnp.zeros((pad,), jnp.int32)])
    wp = jnp.concatenate([edge_weight, jnp.zeros((pad,), jnp.float32)])
    col2 = colp.reshape(EP // CH, CH)

    degp = _deg_sc(col2, wp)
    H = _dense_tc(x, W1, b1, W2, b2.reshape(1, KOUT)).reshape(N, F)
    X1 = _pre_tc(degp, H, Wc1)
    P1 = _conv_sc(X1.reshape(4 * N, QF), rowp, colp, wp)
    X2 = _mid_tc(P1, X1, degp, bc1, H, Wc2)
    P2 = _conv_sc(X2.reshape(4 * N, QF), rowp, colp, wp)
    return _post_tc(P2, X2, degp, bc2, H)


# speedup vs baseline: 1.0308x; 1.0004x over previous
import functools

import jax
import jax.numpy as jnp
import numpy as np
from jax import lax
from jax.experimental import pallas as pl
from jax.experimental.pallas import tpu as pltpu
from jax.experimental.pallas import tpu_sc as plsc

N = 50000
E = 800000
F = 64
HF = 32
G = 20
BN = 128
FFN = 128
KOUT = 160000

NSC = 2
NT = 16
CH = 128
NCH = 56
BLKE = NCH * CH

EPT = 50176
CPT = EPT // CH
NBLK = CPT // NCH
EP = NT * EPT
SR = 8
SLAB = SR * CH
NSL = NCH // SR

NP1 = 50176
ZPT = NP1 // NT
ZR = 392

_mesh = plsc.VectorSubcoreMesh(core_axis_name="c", subcore_axis_name="s")
_sc_params = pltpu.CompilerParams(
    needs_layout_passes=False, use_tc_tiling_on_sc=False
)


def _elu(v):
    return jnp.where(v > 0, v, jnp.exp(v) - 1.0)


@functools.partial(
    pl.kernel,
    out_type=jax.ShapeDtypeStruct((NSC * NP1,), jnp.float32),
    mesh=_mesh,
    scratch_types=[
        pltpu.VMEM((NCH, CH), jnp.int32),
        pltpu.VMEM((BLKE,), jnp.float32),
        pltpu.VMEM((ZPT,), jnp.float32),
        pltpu.VMEM_SHARED((NP1,), jnp.float32),
    ],
)
def _deg_sc(col2_hbm, w_hbm, out_hbm, col_v, w_v, z_v, acc_sh):
    c = lax.axis_index("c")
    s = lax.axis_index("s")
    z16 = jnp.zeros((16,), jnp.float32)

    def zfill(i, _):
        z_v[pl.ds(i * 16, 16)] = z16
        return 0

    lax.fori_loop(0, ZPT // 16, zfill, 0)
    pltpu.sync_copy(z_v, acc_sh.at[pl.ds(s * ZPT, ZPT)])
    plsc.subcore_barrier()

    def blk(b, _):
        crb = s * CPT + b * NCH
        pltpu.sync_copy(col2_hbm.at[pl.ds(crb, NCH)], col_v)
        pltpu.sync_copy(w_hbm.at[pl.ds(crb * CH, BLKE)], w_v)

        def ch(j, _):
            pltpu.sync_copy(
                w_v.at[pl.ds(j * CH, CH)], acc_sh.at[col_v.at[j]], add=True
            )
            return 0

        lax.fori_loop(0, NCH, ch, 0)
        return 0

    lax.fori_loop(0, NBLK, blk, 0)
    plsc.subcore_barrier()
    pltpu.sync_copy(acc_sh.at[pl.ds(s * ZPT, ZPT)], z_v)
    pltpu.sync_copy(z_v, out_hbm.at[pl.ds(c * NP1 + s * ZPT, ZPT)])


QF = 16

_GDN = lax.GatherDimensionNumbers(
    offset_dims=(), collapsed_slice_dims=(0,), start_index_map=(0,)
)


@functools.partial(
    pl.kernel,
    out_type=jax.ShapeDtypeStruct((4, NP1, QF), jnp.float32),
    mesh=_mesh,
    scratch_types=[
        pltpu.VMEM((BLKE,), jnp.int32),
        pltpu.VMEM((BLKE,), jnp.int32),
        pltpu.VMEM((BLKE,), jnp.float32),
        pltpu.VMEM((SLAB, QF), jnp.float32),
        pltpu.VMEM((SLAB, QF), jnp.float32),
        pltpu.VMEM((ZR, QF), jnp.float32),
        pltpu.VMEM_SHARED((NP1, QF), jnp.float32),
        pltpu.SemaphoreType.DMA,
        pltpu.SemaphoreType.DMA,
        pltpu.SemaphoreType.DMA,
        pltpu.SemaphoreType.DMA,
    ],
    compiler_params=_sc_params,
)
def _conv_sc(xs_hbm, row1_hbm, col1_hbm, w_hbm, out_hbm,
             row_v, col_v, w_v, g_a, g_b, z_v, acc_sh,
             sem_a, sem_b, sem_sa, sem_sb):
    c = lax.axis_index("c")
    s = lax.axis_index("s")
    z16 = jnp.zeros((16,), jnp.float32)

    def zfill(i, _):
        z_v[i, pl.ds(0, 16)] = z16
        return 0

    def sweep(p, _):
        q = c * 2 + p
        lax.fori_loop(0, ZR, zfill, 0)

        def zcopy(k, _):
            pltpu.sync_copy(z_v, acc_sh.at[pl.ds(s * ZPT + k * ZR, ZR)])
            return 0

        lax.fori_loop(0, ZPT // ZR, zcopy, 0)
        plsc.subcore_barrier()

        qNv = jnp.full((16,), q * N, jnp.int32)

        def blk(b, _):
            eb = (s * CPT + b * NCH) * CH
            pltpu.sync_copy(row1_hbm.at[pl.ds(eb, BLKE)], row_v)
            pltpu.sync_copy(col1_hbm.at[pl.ds(eb, BLKE)], col_v)
            pltpu.sync_copy(w_hbm.at[pl.ds(eb, BLKE)], w_v)

            def adj(i, _):
                sl = pl.ds(i * 16, 16)
                row_v[sl] = row_v[sl] + qNv
                return 0

            lax.fori_loop(0, BLKE // 16, adj, 0, unroll=8)

            def scale(j, g):
                def grp(g8, _):
                    w16 = w_v[pl.ds(j * SLAB + g8 * 16, 16)]
                    e0 = g8 * 16
                    for k in range(16):
                        wb = lax.gather(
                            w16, jnp.full((16, 1), k, jnp.int32), _GDN, (1,),
                            mode=lax.GatherScatterMode.PROMISE_IN_BOUNDS,
                        )
                        g[e0 + k, pl.ds(0, 16)] = g[e0 + k, pl.ds(0, 16)] * wb
                    return 0

                lax.fori_loop(0, SLAB // 16, grp, 0, unroll=2)

            def gather(j, g, sem):
                pltpu.async_copy(
                    xs_hbm.at[row_v.at[pl.ds(j * SLAB, SLAB)]], g, sem
                )

            def gwait(j, g, sem):
                pltpu.make_async_copy(
                    xs_hbm.at[row_v.at[pl.ds(j * SLAB, SLAB)]], g, sem
                ).wait()

            def scatter(j, g, sem):
                pltpu.async_copy(
                    g, acc_sh.at[col_v.at[pl.ds(j * SLAB, SLAB)]], sem, add=True
                )

            def swait(j, g, sem):
                pltpu.make_async_copy(
                    g, acc_sh.at[col_v.at[pl.ds(j * SLAB, SLAB)]], sem
                ).wait()

            gather(0, g_a, sem_a)

            def slab_loop(j, _):
                @pl.when(j % 2 == 0)
                def _():
                    gwait(j, g_a, sem_a)

                    @pl.when(j < NSL - 1)
                    def _():
                        @pl.when(j >= 1)
                        def _():
                            swait(j - 1, g_b, sem_sb)

                        gather(j + 1, g_b, sem_b)

                    scale(j, g_a)
                    scatter(j, g_a, sem_sa)

                @pl.when(j % 2 == 1)
                def _():
                    gwait(j, g_b, sem_b)

                    @pl.when(j < NSL - 1)
                    def _():
                        swait(j - 1, g_a, sem_sa)
                        gather(j + 1, g_a, sem_a)

                    scale(j, g_b)
                    scatter(j, g_b, sem_sb)

                return 0

            lax.fori_loop(0, NSL, slab_loop, 0)
            swait(NSL - 2, g_b, sem_sb)
            swait(NSL - 1, g_a, sem_sa)
            return 0

        lax.fori_loop(0, NBLK, blk, 0)
        plsc.subcore_barrier()

        def ocopy(k, _):
            sl = pl.ds(s * ZPT + k * ZR, ZR)
            pltpu.sync_copy(acc_sh.at[sl], z_v)
            pltpu.sync_copy(z_v, out_hbm.at[q, sl])
            return 0

        lax.fori_loop(0, ZPT // ZR, ocopy, 0)
        return 0

    lax.fori_loop(0, 2, sweep, 0)


_BW = 1280
_R = 2048
_NRB = (N + _R - 1) // _R


def _dense_body(x_ref, W1_ref, b1_ref, W2_ref, b2_ref, o_ref, h0_ref):
    @pl.when(pl.program_id(0) == 0)
    def _():
        h0 = jnp.dot(x_ref[...], W1_ref[...], preferred_element_type=jnp.float32)
        h0_ref[...] = _elu(h0 + b1_ref[...])

    h = jnp.dot(h0_ref[...], W2_ref[...], preferred_element_type=jnp.float32)
    o_ref[...] = _elu(h + b2_ref[...])


_dense_tc = pl.pallas_call(
    _dense_body,
    grid=(KOUT // _BW,),
    in_specs=[
        pl.BlockSpec((G, BN), lambda i: (0, 0)),
        pl.BlockSpec((BN, FFN), lambda i: (0, 0)),
        pl.BlockSpec((FFN,), lambda i: (0,)),
        pl.BlockSpec((FFN, _BW), lambda i: (0, i)),
        pl.BlockSpec((1, _BW), lambda i: (0, i)),
    ],
    out_specs=pl.BlockSpec((G, _BW), lambda i: (0, i)),
    out_shape=jax.ShapeDtypeStruct((G, KOUT), jnp.float32),
    scratch_shapes=[pltpu.VMEM((G, FFN), jnp.float32)],
)


def _dis_block(deg_ref):
    dis = lax.rsqrt(1.0 + deg_ref[...])
    return lax.broadcast_in_dim(dis, (_R, F), (0,))


def _split_q(o_ref, xs):
    for q in range(4):
        o_ref[q, :, :] = xs[:, q * QF:(q + 1) * QF]


def _cat_q(ref):
    return jnp.concatenate([ref[q] for q in range(4)], axis=1)


def _pre_body(deg_ref, h_ref, Wc_ref, o_ref):
    disb = _dis_block(deg_ref)
    xh = jnp.dot(h_ref[...], Wc_ref[...], preferred_element_type=jnp.float32)
    _split_q(o_ref, disb * xh)


_pre_tc = pl.pallas_call(
    _pre_body,
    grid=(_NRB,),
    in_specs=[
        pl.BlockSpec((_R,), lambda i: (i,)),
        pl.BlockSpec((_R, F), lambda i: (i, 0)),
        pl.BlockSpec((F, F), lambda i: (0, 0)),
    ],
    out_specs=pl.BlockSpec((4, _R, QF), lambda i: (0, i, 0)),
    out_shape=jax.ShapeDtypeStruct((4, N, QF), jnp.float32),
)


def _mid_body(p_ref, xs_ref, deg_ref, bc_ref, h_ref, Wc_ref, o_ref):
    disb = _dis_block(deg_ref)
    acc = _cat_q(p_ref)
    xs = _cat_q(xs_ref)
    y = _elu(disb * (acc + xs) + bc_ref[...]) + h_ref[...]
    xh2 = jnp.dot(y, Wc_ref[...], preferred_element_type=jnp.float32)
    _split_q(o_ref, disb * xh2)


_mid_tc = pl.pallas_call(
    _mid_body,
    grid=(_NRB,),
    in_specs=[
        pl.BlockSpec((4, _R, QF), lambda i: (0, i, 0)),
        pl.BlockSpec((4, _R, QF), lambda i: (0, i, 0)),
        pl.BlockSpec((_R,), lambda i: (i,)),
        pl.BlockSpec((F,), lambda i: (0,)),
        pl.BlockSpec((_R, F), lambda i: (i, 0)),
        pl.BlockSpec((F, F), lambda i: (0, 0)),
    ],
    out_specs=pl.BlockSpec((4, _R, QF), lambda i: (0, i, 0)),
    out_shape=jax.ShapeDtypeStruct((4, N, QF), jnp.float32),
)


def _post_body(p_ref, xs_ref, deg_ref, bc_ref, h_ref, o_ref):
    disb = _dis_block(deg_ref)
    acc = _cat_q(p_ref)
    xs = _cat_q(xs_ref)
    o_ref[...] = disb * (acc + xs) + bc_ref[...] + h_ref[...]


_post_tc = pl.pallas_call(
    _post_body,
    grid=(_NRB,),
    in_specs=[
        pl.BlockSpec((4, _R, QF), lambda i: (0, i, 0)),
        pl.BlockSpec((4, _R, QF), lambda i: (0, i, 0)),
        pl.BlockSpec((_R,), lambda i: (i,)),
        pl.BlockSpec((F,), lambda i: (0,)),
        pl.BlockSpec((_R, F), lambda i: (i, 0)),
    ],
    out_specs=pl.BlockSpec((_R, F), lambda i: (i, 0)),
    out_shape=jax.ShapeDtypeStruct((N, F), jnp.float32),
)


def kernel(x, edge_index, edge_weight, W1, b1, W2, b2, Wc1, bc1, Wc2, bc2,
           num_graphs):
    del num_graphs
    ei = edge_index.astype(jnp.int32)
    pad = EP - E
    rowp = jnp.concatenate([ei[0], jnp.zeros((pad,), jnp.int32)])
    colp = jnp.concatenate([ei[1], jnp.zeros((pad,), jnp.int32)])
    wp = jnp.concatenate([edge_weight, jnp.zeros((pad,), jnp.float32)])
    col2 = colp.reshape(EP // CH, CH)

    degp = _deg_sc(col2, wp)
    H = _dense_tc(x, W1, b1, W2, b2.reshape(1, KOUT)).reshape(N, F)
    X1 = _pre_tc(degp, H, Wc1)
    P1 = _conv_sc(X1.reshape(4 * N, QF), rowp, colp, wp)
    X2 = _mid_tc(P1, X1, degp, bc1, H, Wc2)
    P2 = _conv_sc(X2.reshape(4 * N, QF), rowp, colp, wp)
    return _post_tc(P2, X2, degp, bc2, H)
